# Initial kernel scaffold; baseline (speedup 1.0000x reference)
import jax
import jax.numpy as jnp
from jax.experimental import pallas as pl


def kernel(inputs, tables):
    raise NotImplementedError("write your pallas kernel here")


# speedup vs baseline: 1.0273x; 1.0273x over previous
import functools

import jax
import jax.numpy as jnp
from jax import lax
from jax.experimental import pallas as pl
from jax.experimental.pallas import tpu as pltpu
from jax.experimental.pallas import tpu_sc as plsc

_N_SPARSE = 26
_N_DENSE = 13
_VOCAB = 100000
_EMB = 64
_BATCH = 4096
_NC, _NS = 2, 16
_NW = _NC * _NS
_BPW = _BATCH // _NW
_OUT_D = _N_SPARSE * _EMB + _N_DENSE

_mesh = plsc.VectorSubcoreMesh(
    core_axis_name="c", subcore_axis_name="s",
    num_cores=_NC, num_subcores=_NS,
)


@functools.partial(
    pl.kernel,
    out_type=jax.ShapeDtypeStruct((_BATCH, _OUT_D), jnp.float32),
    mesh=_mesh,
    scratch_types=[
        pltpu.VMEM((_BPW,), jnp.int32),
        pltpu.VMEM((_BPW, _EMB), jnp.float32),
        pltpu.VMEM((_BPW, _N_DENSE), jnp.float32),
        pltpu.SemaphoreType.DMA,
    ],
    compiler_params=pltpu.CompilerParams(use_tc_tiling_on_sc=False),
)
def _sc_embed(tab_hbm, idx_hbm, dense_hbm, out_hbm, idx_v, rows_v, dense_v, sem):
    wid = lax.axis_index("s") * _NC + lax.axis_index("c")
    base = wid * _BPW

    pltpu.sync_copy(dense_hbm.at[pl.ds(base, _BPW), :], dense_v)
    pltpu.sync_copy(dense_v,
                    out_hbm.at[pl.ds(base, _BPW),
                               pl.ds(_N_SPARSE * _EMB, _N_DENSE)])

    def body(i, carry):
        pltpu.sync_copy(idx_hbm.at[i, pl.ds(base, _BPW)], idx_v)
        pltpu.async_copy(tab_hbm.at[idx_v], rows_v, sem).wait()
        pltpu.sync_copy(rows_v,
                        out_hbm.at[pl.ds(base, _BPW), pl.ds(i * _EMB, _EMB)])
        return carry

    lax.fori_loop(0, _N_SPARSE, body, 0)


def kernel(inputs, tables):
    sp = inputs[:, :_N_SPARSE].astype(jnp.int32)
    gidx = (jnp.transpose(sp)
            + (jnp.arange(_N_SPARSE, dtype=jnp.int32) * _VOCAB)[:, None])
    tab_flat = tables.reshape(_N_SPARSE * _VOCAB, _EMB)
    dense = inputs[:, _N_SPARSE:]
    return _sc_embed(tab_flat, gidx, dense)


# speedup vs baseline: 1.0354x; 1.0079x over previous
import functools

import jax
import jax.numpy as jnp
from jax import lax
from jax.experimental import pallas as pl
from jax.experimental.pallas import tpu as pltpu
from jax.experimental.pallas import tpu_sc as plsc

_N_SPARSE = 26
_N_DENSE = 13
_VOCAB = 100000
_EMB = 64
_BATCH = 4096
_NC, _NS = 2, 16
_NW = _NC * _NS
_BPW = _BATCH // _NW
_IPW = _N_SPARSE * _BPW
_OUT_D = _N_SPARSE * _EMB + _N_DENSE

_mesh = plsc.VectorSubcoreMesh(
    core_axis_name="c", subcore_axis_name="s",
    num_cores=_NC, num_subcores=_NS,
)


@functools.partial(
    pl.kernel,
    out_type=jax.ShapeDtypeStruct((_BATCH, _OUT_D), jnp.float32),
    mesh=_mesh,
    scratch_types=[
        pltpu.VMEM((_IPW,), jnp.int32),
        pltpu.VMEM((_BPW, _EMB), jnp.float32),
        pltpu.VMEM((_BPW, _N_DENSE), jnp.float32),
        pltpu.SemaphoreType.DMA,
    ],
    compiler_params=pltpu.CompilerParams(use_tc_tiling_on_sc=False),
)
def _sc_embed(tab_hbm, idx_hbm, dense_hbm, out_hbm, idx_v, rows_v, dense_v, sem):
    wid = lax.axis_index("s") * _NC + lax.axis_index("c")
    base = wid * _BPW

    pltpu.sync_copy(idx_hbm.at[pl.ds(wid * _IPW, _IPW)], idx_v)

    pltpu.sync_copy(dense_hbm.at[pl.ds(base, _BPW), :], dense_v)
    pltpu.sync_copy(dense_v,
                    out_hbm.at[pl.ds(base, _BPW),
                               pl.ds(_N_SPARSE * _EMB, _N_DENSE)])

    def body(i, carry):
        pltpu.async_copy(tab_hbm.at[idx_v.at[pl.ds(i * _BPW, _BPW)]],
                         rows_v, sem).wait()
        pltpu.sync_copy(rows_v,
                        out_hbm.at[pl.ds(base, _BPW), pl.ds(i * _EMB, _EMB)])
        return carry

    lax.fori_loop(0, _N_SPARSE, body, 0)


def kernel(inputs, tables):
    sp = inputs[:, :_N_SPARSE].astype(jnp.int32)
    gidx = (jnp.transpose(sp)
            + (jnp.arange(_N_SPARSE, dtype=jnp.int32) * _VOCAB)[:, None])
    idx1d = gidx.reshape(_N_SPARSE, _NW, _BPW).transpose(1, 0, 2).reshape(-1)
    tab_flat = tables.reshape(_N_SPARSE * _VOCAB, _EMB)
    dense = inputs[:, _N_SPARSE:]
    return _sc_embed(tab_flat, idx1d, dense)
